# baseline (speedup 1.0000x reference)
import functools

import jax
import jax.numpy as jnp
from jax import lax
from jax.experimental import pallas as pl
from jax.experimental.pallas import tpu as pltpu
from jax.experimental.pallas import tpu_sc as plsc

_NUM_TOKENS = 8192
_D = 32
_B = 64
_T = 1024
_N = _B * _T
_NC = 2
_NS = 16
_NW = _NC * _NS
_PER_W = _N // _NW
_ROWS_W = _B // _NW
_CHUNK = 128
_NCHUNK = _PER_W // _CHUNK
_CPR = _T // _CHUNK
_NBUF = 4
_L = 16

_mesh = plsc.VectorSubcoreMesh(core_axis_name="c", subcore_axis_name="s")


@functools.partial(
    pl.kernel,
    mesh=_mesh,
    out_type=jax.ShapeDtypeStruct((_N * _D,), jnp.float32),
    scratch_types=[
        pltpu.VMEM((_ROWS_W, _T), jnp.int32),
        pltpu.VMEM((_NBUF, _CHUNK, _D), jnp.float32),
        pltpu.VMEM((_PER_W * _D,), jnp.float32),
        pltpu.SemaphoreType.DMA,
    ],
    compiler_params=pltpu.CompilerParams(
        use_tc_tiling_on_sc=False,
        needs_layout_passes=False,
        disable_bounds_checks=True,
    ),
)
def _gather_kernel(idx_hbm, table_hbm, out_hbm, idx_v, bufs, packs, gsem):
    wid = lax.axis_index("s") * _NC + lax.axis_index("c")
    pltpu.sync_copy(idx_hbm.at[pl.ds(wid * _ROWS_W, _ROWS_W)], idx_v)

    lanes = lax.iota(jnp.int32, _L)
    col_a = lanes * _T
    col_b = (lanes + _L) * _T

    def fire_gather(j):
        idx_row = idx_v.at[j // _CPR, pl.ds((j % _CPR) * _CHUNK, _CHUNK)]
        return pltpu.async_copy(
            table_hbm.at[idx_row], bufs.at[j % _NBUF], gsem
        )

    def repack(j):
        jb = j % _NBUF
        base = (j // _CPR) * _D * _T + (j % _CPR) * _CHUNK

        @plsc.parallel_loop(0, _CHUNK, unroll=4)
        def body(t):
            v0 = bufs[jb, t, pl.ds(0, _L)]
            v1 = bufs[jb, t, pl.ds(_L, _L)]
            tb = base + t
            plsc.store_scatter(packs, [col_a + tb], v0)
            plsc.store_scatter(packs, [col_b + tb], v1)

    gathers = [None] * _NCHUNK
    for j in range(_NBUF):
        gathers[j] = fire_gather(j)
    for j in range(_NCHUNK):
        gathers[j].wait()
        repack(j)
        if j + _NBUF < _NCHUNK:
            gathers[j + _NBUF] = fire_gather(j + _NBUF)
    pltpu.sync_copy(packs, out_hbm.at[pl.ds(wid * _PER_W * _D, _PER_W * _D)])


def kernel(embed_id, weight):
    out_t = _gather_kernel(embed_id, weight)
    return out_t.reshape(_B, _D, _T).transpose(0, 2, 1)


# speedup vs baseline: 1.0592x; 1.0592x over previous
import functools

import jax
import jax.numpy as jnp
from jax import lax
from jax.experimental import pallas as pl
from jax.experimental.pallas import tpu as pltpu
from jax.experimental.pallas import tpu_sc as plsc

_NUM_TOKENS = 8192
_D = 32
_B = 64
_T = 1024
_N = _B * _T
_NC = 2
_NS = 16
_NW = _NC * _NS
_PER_W = _N // _NW
_ROWS_W = _B // _NW
_CHUNK = 128
_NCHUNK = _PER_W // _CHUNK
_CPR = _T // _CHUNK
_NBUF = 4
_L = 16

_mesh = plsc.VectorSubcoreMesh(core_axis_name="c", subcore_axis_name="s")


@functools.partial(
    pl.kernel,
    mesh=_mesh,
    out_type=jax.ShapeDtypeStruct((_B, _D, _T), jnp.float32),
    scratch_types=[
        pltpu.VMEM((_ROWS_W, _T), jnp.int32),
        pltpu.VMEM((_NBUF, _CHUNK, _D), jnp.float32),
        pltpu.VMEM((_ROWS_W, _D, _T), jnp.float32),
        pltpu.SemaphoreType.DMA,
    ],
    compiler_params=pltpu.CompilerParams(
        use_tc_tiling_on_sc=False,
        needs_layout_passes=False,
        disable_bounds_checks=True,
    ),
)
def _gather_kernel(idx_hbm, table_hbm, out_hbm, idx_v, bufs, packs, gsem):
    wid = lax.axis_index("s") * _NC + lax.axis_index("c")
    pltpu.sync_copy(idx_hbm.at[pl.ds(wid * _ROWS_W, _ROWS_W)], idx_v)

    lanes = lax.iota(jnp.int32, _L)
    t0vs = [lanes + g * _L for g in range(_CHUNK // _L)]

    def fire_gather(j):
        idx_row = idx_v.at[j // _CPR, pl.ds((j % _CPR) * _CHUNK, _CHUNK)]
        return pltpu.async_copy(
            table_hbm.at[idx_row], bufs.at[j % _NBUF], gsem
        )

    def repack(j):
        jb = j % _NBUF
        jr = j // _CPR
        c0 = (j % _CPR) * _CHUNK

        @plsc.parallel_loop(0, _D, unroll=4)
        def body(d):
            dspl = jnp.broadcast_to(d, (_L,)).astype(jnp.int32)
            for g in range(_CHUNK // _L):
                v = plsc.load_gather(bufs.at[jb], [t0vs[g], dspl])
                packs[jr, d, pl.ds(c0 + g * _L, _L)] = v

    gathers = [None] * _NCHUNK
    for j in range(_NBUF):
        gathers[j] = fire_gather(j)
    for j in range(_NCHUNK):
        gathers[j].wait()
        repack(j)
        if j + _NBUF < _NCHUNK:
            gathers[j + _NBUF] = fire_gather(j + _NBUF)
    pltpu.sync_copy(packs, out_hbm.at[pl.ds(wid * _ROWS_W, _ROWS_W)])


def kernel(embed_id, weight):
    out_t = _gather_kernel(embed_id, weight)
    return out_t.transpose(0, 2, 1)
